# Initial kernel scaffold; baseline (speedup 1.0000x reference)
import jax
import jax.numpy as jnp
from jax.experimental import pallas as pl


def kernel(x, position_ids, cos_cached, sin_cached):
    raise NotImplementedError("write your pallas kernel here")


# speedup vs baseline: 4.8739x; 4.8739x over previous
import functools

import jax
import jax.numpy as jnp
from jax import lax
from jax.experimental import pallas as pl
from jax.experimental.pallas import tpu as pltpu
from jax.experimental.pallas import tpu_sc as plsc

DIM = 128
CHUNK = 128


@functools.lru_cache(maxsize=None)
def _make_gather(n_idx):
    info = plsc.get_sparse_core_info()
    nc, ns = info.num_cores, info.num_subcores
    nw = nc * ns
    b_per_w = n_idx // nw
    n_chunks = b_per_w // CHUNK
    mesh = plsc.VectorSubcoreMesh(core_axis_name="c", subcore_axis_name="s")

    @functools.partial(
        pl.kernel,
        out_type=(
            jax.ShapeDtypeStruct((n_idx, DIM), jnp.float32),
            jax.ShapeDtypeStruct((n_idx, DIM), jnp.float32),
        ),
        mesh=mesh,
        scratch_types=[
            pltpu.VMEM((n_chunks, CHUNK), jnp.int32),
            pltpu.VMEM((CHUNK, DIM), jnp.float32),
            pltpu.VMEM((CHUNK, DIM), jnp.float32),
            pltpu.SemaphoreType.DMA,
            pltpu.SemaphoreType.DMA,
        ],
    )
    def gather_kernel(pos_hbm, cos_hbm, sin_hbm, cos_out, sin_out,
                      idx_v, cbuf, sbuf, csem, ssem):
        wid = lax.axis_index("s") * nc + lax.axis_index("c")
        base = wid * b_per_w
        pltpu.sync_copy(pos_hbm.at[pl.ds(wid * n_chunks, n_chunks)], idx_v)
        for j in range(n_chunks):
            ccopy = pltpu.async_copy(cos_hbm.at[idx_v.at[j]], cbuf, csem)
            scopy = pltpu.async_copy(sin_hbm.at[idx_v.at[j]], sbuf, ssem)
            ccopy.wait()
            pltpu.sync_copy(cbuf, cos_out.at[pl.ds(base + j * CHUNK, CHUNK)])
            scopy.wait()
            pltpu.sync_copy(sbuf, sin_out.at[pl.ds(base + j * CHUNK, CHUNK)])

    return gather_kernel


def kernel(x, position_ids, cos_cached, sin_cached):
    del x
    b, s = position_ids.shape
    n = b * s
    pos2d = position_ids.reshape(n // CHUNK, CHUNK)
    cos, sin = _make_gather(n)(pos2d, cos_cached, sin_cached)
    return cos.reshape(b, s, DIM), sin.reshape(b, s, DIM)


# speedup vs baseline: 5.1958x; 1.0660x over previous
import functools

import jax
import jax.numpy as jnp
from jax import lax
from jax.experimental import pallas as pl
from jax.experimental.pallas import tpu as pltpu
from jax.experimental.pallas import tpu_sc as plsc

DIM = 128
CHUNK = 128


@functools.lru_cache(maxsize=None)
def _make_gather(n_idx):
    info = plsc.get_sparse_core_info()
    nc, ns = info.num_cores, info.num_subcores
    nw = nc * ns
    b_per_w = n_idx // nw
    n_chunks = b_per_w // CHUNK
    mesh = plsc.VectorSubcoreMesh(core_axis_name="c", subcore_axis_name="s")

    @functools.partial(
        pl.kernel,
        out_type=(
            jax.ShapeDtypeStruct((n_idx, DIM), jnp.float32),
            jax.ShapeDtypeStruct((n_idx, DIM), jnp.float32),
        ),
        mesh=mesh,
        scratch_types=[
            pltpu.VMEM((n_chunks, CHUNK), jnp.int32),
            pltpu.VMEM((2, CHUNK, DIM), jnp.float32),
            pltpu.VMEM((2, CHUNK, DIM), jnp.float32),
        ] + [pltpu.SemaphoreType.DMA] * 8,
    )
    def gather_kernel(pos_hbm, cos_hbm, sin_hbm, cos_out, sin_out,
                      idx_v, cbuf, sbuf,
                      cg0, cg1, sg0, sg1, co0, co1, so0, so1):
        cg = (cg0, cg1)
        sg = (sg0, sg1)
        co = (co0, co1)
        so = (so0, so1)
        wid = lax.axis_index("s") * nc + lax.axis_index("c")
        base = wid * b_per_w
        pltpu.sync_copy(pos_hbm.at[pl.ds(wid * n_chunks, n_chunks)], idx_v)
        c_g = [None, None]
        s_g = [None, None]
        c_o = [None, None]
        s_o = [None, None]
        c_g[0] = pltpu.async_copy(cos_hbm.at[idx_v.at[0]], cbuf.at[0], cg[0])
        s_g[0] = pltpu.async_copy(sin_hbm.at[idx_v.at[0]], sbuf.at[0], sg[0])
        for j in range(n_chunks):
            b = j & 1
            nb = (j + 1) & 1
            if j + 1 < n_chunks:
                if c_o[nb] is not None:
                    c_o[nb].wait()
                c_g[nb] = pltpu.async_copy(
                    cos_hbm.at[idx_v.at[j + 1]], cbuf.at[nb], cg[nb])
                if s_o[nb] is not None:
                    s_o[nb].wait()
                s_g[nb] = pltpu.async_copy(
                    sin_hbm.at[idx_v.at[j + 1]], sbuf.at[nb], sg[nb])
            c_g[b].wait()
            c_o[b] = pltpu.async_copy(
                cbuf.at[b], cos_out.at[pl.ds(base + j * CHUNK, CHUNK)], co[b])
            s_g[b].wait()
            s_o[b] = pltpu.async_copy(
                sbuf.at[b], sin_out.at[pl.ds(base + j * CHUNK, CHUNK)], so[b])
        c_o[0].wait()
        c_o[1].wait()
        s_o[0].wait()
        s_o[1].wait()

    return gather_kernel


def kernel(x, position_ids, cos_cached, sin_cached):
    del x
    b, s = position_ids.shape
    n = b * s
    pos2d = position_ids.reshape(n // CHUNK, CHUNK)
    cos, sin = _make_gather(n)(pos2d, cos_cached, sin_cached)
    return cos.reshape(b, s, DIM), sin.reshape(b, s, DIM)
